# baseline (speedup 1.0000x reference)
import functools

import jax
import jax.numpy as jnp
from jax import lax
from jax.experimental import pallas as pl
from jax.experimental.pallas import tpu as pltpu
from jax.experimental.pallas import tpu_sc as plsc

N_ROWS = 1000000
FEATURES = 64
BATCH = 16384

_info = plsc.get_sparse_core_info()
_NC = _info.num_cores
_NS = _info.num_subcores
_NW = _NC * _NS
assert _NW == 32

_B_PER_W = BATCH // _NW
_CHUNK = 128
_NCHUNK = _B_PER_W // _CHUNK

_BLK = 4096
_SH = _BLK.bit_length() - 1
_HM = _BLK // 2 - 1
_NBLK = (N_ROWS + _BLK - 1) // _BLK
_PAIR_ROWS = _NBLK * _BLK // 2

_mesh = plsc.VectorSubcoreMesh(core_axis_name="c", subcore_axis_name="s")


def _relayout_body(w_ref, out_ref):
    eye = jnp.float32(
        lax.broadcasted_iota(jnp.int32, (FEATURES, FEATURES), 0)
        == lax.broadcasted_iota(jnp.int32, (FEATURES, FEATURES), 1)
    )
    t = lax.dot_general(
        w_ref[...], eye, (((0,), (0,)), ((), ())),
        preferred_element_type=jnp.float32,
        precision=lax.Precision.HIGHEST,
    )
    out_ref[0] = jnp.concatenate([t[: _BLK // 2], t[_BLK // 2 :]], axis=1)


_relayout = pl.pallas_call(
    _relayout_body,
    grid=(_NBLK,),
    in_specs=[pl.BlockSpec((FEATURES, _BLK), lambda i: (0, i))],
    out_specs=pl.BlockSpec((1, _BLK // 2, 2 * FEATURES), lambda i: (i, 0, 0)),
    out_shape=jax.ShapeDtypeStruct((_NBLK, _BLK // 2, 2 * FEATURES),
                                   jnp.float32),
)


@functools.partial(
    pl.kernel,
    mesh=_mesh,
    out_type=jax.ShapeDtypeStruct((BATCH, 2 * FEATURES), jnp.float32),
    scratch_types=[
        pltpu.VMEM((_NCHUNK, _CHUNK), jnp.int32),
        pltpu.VMEM((_NCHUNK, _CHUNK), jnp.int32),
        pltpu.VMEM((_NCHUNK, _CHUNK, 2 * FEATURES), jnp.float32),
        pltpu.SemaphoreType.DMA,
        pltpu.SemaphoreType.DMA,
        pltpu.SemaphoreType.DMA,
    ],
)
def _gather_pairs(wt_hbm, idx_hbm, stage_hbm, idx_v, pidx_v, pair_v,
                  sem_i, sem_g, sem_o):
    w = lax.axis_index("s") * _NC + lax.axis_index("c")
    base = pl.multiple_of(w * _B_PER_W, _B_PER_W)

    for j in range(_NCHUNK):
        pltpu.make_async_copy(
            idx_hbm.at[pl.ds(base + _CHUNK * j, _CHUNK)],
            idx_v.at[j],
            sem_i,
        ).start()
    for j in range(_NCHUNK):
        pltpu.make_async_copy(
            idx_hbm.at[pl.ds(0, _CHUNK)], idx_v.at[j], sem_i
        ).wait()

    for j in range(_NCHUNK):
        @pl.loop(0, _CHUNK // 16)
        def _pid(q):
            m0 = pl.multiple_of(q * 16, 16)
            iv = idx_v[j, pl.ds(m0, 16)]
            pidx_v[j, pl.ds(m0, 16)] = ((iv >> _SH) << (_SH - 1)) | (iv & _HM)

    for j in range(_NCHUNK):
        pltpu.make_async_copy(
            wt_hbm.at[pidx_v.at[j]], pair_v.at[j], sem_g
        ).start()
    for j in range(_NCHUNK):
        pltpu.make_async_copy(
            wt_hbm.at[pidx_v.at[j]], pair_v.at[j], sem_g
        ).wait()

    for j in range(_NCHUNK):
        pltpu.make_async_copy(
            pair_v.at[j],
            stage_hbm.at[pl.ds(base + _CHUNK * j, _CHUNK)],
            sem_o,
        ).start()
    for j in range(_NCHUNK):
        pltpu.make_async_copy(
            pair_v.at[j],
            stage_hbm.at[pl.ds(0, _CHUNK)],
            sem_o,
        ).wait()


_TC_ROWS = 256
_TC_GRID = BATCH // _TC_ROWS


def _select_body(par_ref, stage_ref, out_ref):
    s = stage_ref[0]
    p = par_ref[0] != 0
    out_ref[0] = jnp.where(p, s[:, FEATURES:], s[:, :FEATURES])


_select = pl.pallas_call(
    _select_body,
    grid=(_TC_GRID,),
    in_specs=[
        pl.BlockSpec((1, _TC_ROWS, 1), lambda i: (i, 0, 0)),
        pl.BlockSpec((1, _TC_ROWS, 2 * FEATURES), lambda i: (i, 0, 0)),
    ],
    out_specs=pl.BlockSpec((1, _TC_ROWS, FEATURES), lambda i: (i, 0, 0)),
    out_shape=jax.ShapeDtypeStruct((_TC_GRID, _TC_ROWS, FEATURES),
                                   jnp.float32),
)


@jax.jit
def kernel(idx, weight):
    idx = idx.astype(jnp.int32)
    pairs_tbl = _relayout(weight.T).reshape(_PAIR_ROWS, 2 * FEATURES)
    stage = _gather_pairs(pairs_tbl, idx)
    par = ((idx >> (_SH - 1)) & 1).reshape(_TC_GRID, _TC_ROWS, 1)
    out = _select(par, stage.reshape(_TC_GRID, _TC_ROWS, 2 * FEATURES))
    return out.reshape(BATCH, FEATURES)


# speedup vs baseline: 2.2500x; 2.2500x over previous
import functools

import jax
import jax.numpy as jnp
from jax import lax
from jax.experimental import pallas as pl
from jax.experimental.pallas import tpu as pltpu
from jax.experimental.pallas import tpu_sc as plsc

N_ROWS = 1000000
FEATURES = 64
BATCH = 16384

_info = plsc.get_sparse_core_info()
_NC = _info.num_cores
_NS = _info.num_subcores
_NW = _NC * _NS
assert _NW == 32

_B_PER_W = BATCH // _NW
_CHUNK = 128
_NCHUNK = _B_PER_W // _CHUNK

_BLK = 4096
_SH = _BLK.bit_length() - 1
_HM = _BLK // 2 - 1
_NBLK = (N_ROWS + _BLK - 1) // _BLK
_PAIR_ROWS = _NBLK * _BLK // 2

_mesh = plsc.VectorSubcoreMesh(core_axis_name="c", subcore_axis_name="s")


def _relayout_body(w_ref, out_ref):
    t = w_ref[...].T
    out_ref[0] = jnp.concatenate([t[: _BLK // 2], t[_BLK // 2 :]], axis=1)


_relayout = pl.pallas_call(
    _relayout_body,
    grid=(_NBLK,),
    in_specs=[pl.BlockSpec((FEATURES, _BLK), lambda i: (0, i))],
    out_specs=pl.BlockSpec((1, _BLK // 2, 2 * FEATURES), lambda i: (i, 0, 0)),
    out_shape=jax.ShapeDtypeStruct((_NBLK, _BLK // 2, 2 * FEATURES),
                                   jnp.float32),
)


@functools.partial(
    pl.kernel,
    mesh=_mesh,
    out_type=jax.ShapeDtypeStruct((BATCH, 2 * FEATURES), jnp.float32),
    scratch_types=[
        pltpu.VMEM((_NCHUNK, _CHUNK), jnp.int32),
        pltpu.VMEM((_NCHUNK, _CHUNK), jnp.int32),
        pltpu.VMEM((_NCHUNK, _CHUNK, 2 * FEATURES), jnp.float32),
        pltpu.SemaphoreType.DMA,
        pltpu.SemaphoreType.DMA,
        pltpu.SemaphoreType.DMA,
    ],
)
def _gather_pairs(wt_hbm, idx_hbm, stage_hbm, idx_v, pidx_v, pair_v,
                  sem_i, sem_g, sem_o):
    w = lax.axis_index("s") * _NC + lax.axis_index("c")
    base = pl.multiple_of(w * _B_PER_W, _B_PER_W)

    for j in range(_NCHUNK):
        pltpu.make_async_copy(
            idx_hbm.at[pl.ds(base + _CHUNK * j, _CHUNK)],
            idx_v.at[j],
            sem_i,
        ).start()
    for j in range(_NCHUNK):
        pltpu.make_async_copy(
            idx_hbm.at[pl.ds(0, _CHUNK)], idx_v.at[j], sem_i
        ).wait()

    for j in range(_NCHUNK):
        @pl.loop(0, _CHUNK // 16)
        def _pid(q):
            m0 = pl.multiple_of(q * 16, 16)
            iv = idx_v[j, pl.ds(m0, 16)]
            pidx_v[j, pl.ds(m0, 16)] = ((iv >> _SH) << (_SH - 1)) | (iv & _HM)

    for j in range(_NCHUNK):
        pltpu.make_async_copy(
            wt_hbm.at[pidx_v.at[j]], pair_v.at[j], sem_g
        ).start()
    for j in range(_NCHUNK):
        pltpu.make_async_copy(
            wt_hbm.at[pidx_v.at[j]], pair_v.at[j], sem_g
        ).wait()

    for j in range(_NCHUNK):
        pltpu.make_async_copy(
            pair_v.at[j],
            stage_hbm.at[pl.ds(base + _CHUNK * j, _CHUNK)],
            sem_o,
        ).start()
    for j in range(_NCHUNK):
        pltpu.make_async_copy(
            pair_v.at[j],
            stage_hbm.at[pl.ds(0, _CHUNK)],
            sem_o,
        ).wait()


_TC_ROWS = 256
_TC_GRID = BATCH // _TC_ROWS


def _select_body(par_ref, stage_ref, out_ref):
    s = stage_ref[0]
    p = par_ref[0] != 0
    out_ref[0] = jnp.where(p, s[:, FEATURES:], s[:, :FEATURES])


_select = pl.pallas_call(
    _select_body,
    grid=(_TC_GRID,),
    in_specs=[
        pl.BlockSpec((1, _TC_ROWS, 1), lambda i: (i, 0, 0)),
        pl.BlockSpec((1, _TC_ROWS, 2 * FEATURES), lambda i: (i, 0, 0)),
    ],
    out_specs=pl.BlockSpec((1, _TC_ROWS, FEATURES), lambda i: (i, 0, 0)),
    out_shape=jax.ShapeDtypeStruct((_TC_GRID, _TC_ROWS, FEATURES),
                                   jnp.float32),
)


@jax.jit
def kernel(idx, weight):
    idx = idx.astype(jnp.int32)
    pairs_tbl = _relayout(weight.T).reshape(_PAIR_ROWS, 2 * FEATURES)
    stage = _gather_pairs(pairs_tbl, idx)
    par = ((idx >> (_SH - 1)) & 1).reshape(_TC_GRID, _TC_ROWS, 1)
    out = _select(par, stage.reshape(_TC_GRID, _TC_ROWS, 2 * FEATURES))
    return out.reshape(BATCH, FEATURES)
